# baseline (speedup 1.0000x reference)
import functools
import jax
import jax.numpy as jnp
from jax import lax
from jax.experimental import pallas as pl
from jax.experimental.pallas import tpu as pltpu
from jax.experimental.pallas import tpu_sc as plsc

PLANE_SIZE = 1024
PLANE_EPS = 0.004
BOUND = 2.0
INV_EPS2 = 1.0 / (PLANE_EPS * PLANE_EPS)

NC, NS, L = 2, 16, 16
NW = NC * NS
N = 16384 * 256
PER_W = N // NW
CHUNK = 2048
G = CHUNK // 128
N_CHUNKS = PER_W // CHUNK


def _body(tab_hbm, pos_hbm, w_hbm, out_hbm,
          pos_v0, pos_v1, pos_v2, pos_v3, w_v0, w_v1, w_v2, w_v3,
          zz_v0, zz_v1, idx_v0, idx_v1, g_v0, g_v1,
          out_v0, out_v1, isem, gsem, osem):
    pos_bufs = (pos_v0, pos_v1, pos_v2, pos_v3)
    w_bufs = (w_v0, w_v1, w_v2, w_v3)
    zz_bufs = (zz_v0, zz_v1)
    idx_bufs = (idx_v0, idx_v1)
    g_bufs = (g_v0, g_v1)
    out_bufs = (out_v0, out_v1)
    wid = lax.axis_index("s") * NC + lax.axis_index("c")
    base_w = wid * PER_W
    iota = lax.iota(jnp.int32, L)

    def in_copies(c, q, sp):
        base = base_w + c * CHUNK
        return (pltpu.make_async_copy(
                    pos_hbm.at[pl.ds(base * 3, 3 * CHUNK)], pos_bufs[q],
                    isem.at[sp]),
                pltpu.make_async_copy(
                    w_hbm.at[pl.ds(base, CHUNK)], w_bufs[q], isem.at[sp]))

    def gather_copies(p):
        return (pltpu.make_async_copy(tab_hbm.at[idx_bufs[p]], g_bufs[p],
                                      gsem.at[p]),)

    def out_copy(c, p):
        base = base_w + c * CHUNK
        return pltpu.make_async_copy(out_bufs[p],
                                     out_hbm.at[pl.ds(base, CHUNK)],
                                     osem.at[p])

    def pass_a(q, p):
        def body(i, carry):
            p3 = i * (3 * L) + iota * 3
            x = plsc.load_gather(pos_bufs[q], [p3])
            y = plsc.load_gather(pos_bufs[q], [p3 + 1])
            z = plsc.load_gather(pos_bufs[q], [p3 + 2])
            mag = jnp.maximum(jnp.maximum(jnp.abs(x), jnp.abs(y)), jnp.abs(z))
            r = 1.0 / jnp.maximum(mag, 1e-9)
            s = jnp.where(mag <= 1.0, 1.0, (2.0 - r) * r)
            cz = z * s
            fx = jnp.minimum((x * s + BOUND) * (PLANE_SIZE * 0.5 / BOUND),
                             PLANE_SIZE - 1.0)
            fy = jnp.minimum((y * s + BOUND) * (PLANE_SIZE * 0.5 / BOUND),
                             PLANE_SIZE - 1.0)
            xy2 = (fx.astype(jnp.int32) * PLANE_SIZE
                   + fy.astype(jnp.int32)) * 2
            zz_bufs[p][pl.ds(i * L, L)] = cz
            pos2 = i * (2 * L) + iota * 2
            plsc.store_scatter(idx_bufs[p], [pos2], xy2)
            plsc.store_scatter(idx_bufs[p], [pos2 + 1], xy2 + 1)
            return carry
        lax.fori_loop(0, CHUNK // L, body, None, unroll=2)

    def pass_b(qprev, pprev):
        def body(i, carry):
            j2 = i * (2 * L) + iota * 2
            zmin = plsc.load_gather(g_bufs[pprev], [j2])
            zmax = plsc.load_gather(g_bufs[pprev], [j2 + 1])
            zz = zz_bufs[pprev][pl.ds(i * L, L)]
            w = w_bufs[qprev][pl.ds(i * L, L)]
            wm = jnp.where(zz < zmin, 0.0, 1.0)
            wm = jnp.where(zz > zmax, 0.0, wm)
            d1 = zz - zmin
            d2 = zmax - zz
            q1 = d1 * d1 * INV_EPS2
            q2 = d2 * d2 * INV_EPS2
            i1 = (zz < zmin + PLANE_EPS) & (zz > zmin) & (zz < zmax - PLANE_EPS)
            i2 = (zz > zmax - PLANE_EPS) & (zz < zmax) & (zz > zmin + PLANE_EPS)
            wm = jnp.where(i1, q1, wm)
            wm = jnp.where(i2, q2, wm)
            out_bufs[pprev][pl.ds(i * L, L)] = w * wm
            return carry
        lax.fori_loop(0, CHUNK // L, body, None, unroll=2)

    def tail(cprev, qprev, pprev, wait_out):
        for a in gather_copies(pprev):
            a.wait()
        if wait_out is None:
            out_copy(cprev - 2, pprev).wait()
        else:
            pl.when(wait_out)(
                lambda: out_copy(jnp.maximum(cprev - 2, 0), pprev).wait())
        pass_b(qprev, pprev)
        out_copy(cprev, pprev).start()

    for a in in_copies(0, 0, 0):
        a.start()
    for a in in_copies(1, 1, 1):
        a.start()

    def quad(cc, carry):
        for q in range(4):
            c = 4 * cc + q
            p = q % 2
            for a in in_copies(c, q, p):
                a.wait()
            cnxt = jnp.minimum(c + 2, N_CHUNKS - 1)
            for a in in_copies(cnxt, (q + 2) % 4, p):
                a.start()
            pass_a(q, p)
            for a in gather_copies(p):
                a.start()
            if q == 0:
                pl.when(cc > 0)(
                    lambda: tail(jnp.maximum(c - 1, 0), 3, 1, None))
            elif q == 3:
                tail(c - 1, 2, 0, None)
            else:
                tail(c - 1, q - 1, (q - 1) % 2, cc > 0)
        return carry

    lax.fori_loop(0, N_CHUNKS // 4, quad, None)

    clast = N_CHUNKS - 1
    tail(clast, 3, 1, None)
    out_copy(clast - 1, 0).wait()
    out_copy(clast, 1).wait()
    for a in in_copies(N_CHUNKS - 1, 0, 0):
        a.wait()
    for a in in_copies(N_CHUNKS - 1, 1, 1):
        a.wait()


@jax.jit
def _occ_mask_sc(tab, pos_flat, w_flat):
    mesh = plsc.VectorSubcoreMesh(core_axis_name="c", subcore_axis_name="s",
                                  num_cores=NC, num_subcores=NS)
    f = pl.kernel(
        _body,
        out_type=jax.ShapeDtypeStruct((N,), jnp.float32),
        mesh=mesh,
        compiler_params=pltpu.CompilerParams(needs_layout_passes=False),
        scratch_types=[
            pltpu.VMEM((3 * CHUNK,), jnp.float32),
            pltpu.VMEM((3 * CHUNK,), jnp.float32),
            pltpu.VMEM((3 * CHUNK,), jnp.float32),
            pltpu.VMEM((3 * CHUNK,), jnp.float32),
            pltpu.VMEM((CHUNK,), jnp.float32),
            pltpu.VMEM((CHUNK,), jnp.float32),
            pltpu.VMEM((CHUNK,), jnp.float32),
            pltpu.VMEM((CHUNK,), jnp.float32),
            pltpu.VMEM((CHUNK,), jnp.float32),
            pltpu.VMEM((CHUNK,), jnp.float32),
            pltpu.VMEM((2 * CHUNK,), jnp.int32),
            pltpu.VMEM((2 * CHUNK,), jnp.int32),
            pltpu.VMEM((2 * CHUNK,), jnp.float32),
            pltpu.VMEM((2 * CHUNK,), jnp.float32),
            pltpu.VMEM((CHUNK,), jnp.float32),
            pltpu.VMEM((CHUNK,), jnp.float32),
            pltpu.SemaphoreType.DMA((2,)),
            pltpu.SemaphoreType.DMA((2,)),
            pltpu.SemaphoreType.DMA((2,)),
        ],
    )
    return f(tab, pos_flat, w_flat)


def kernel(positions, weights, occ_plane_min, occ_plane_max):
    prefix = positions.shape[:-1]
    tab = jnp.stack([occ_plane_min, occ_plane_max], axis=1).reshape(-1)
    out = _occ_mask_sc(tab, positions.reshape(-1), weights.reshape(-1))
    return out.reshape(prefix + (1,))


# speedup vs baseline: 1.1867x; 1.1867x over previous
import jax
import jax.numpy as jnp
from jax import lax
from jax.experimental import pallas as pl
from jax.experimental.pallas import tpu as pltpu
from jax.experimental.pallas import tpu_sc as plsc

PLANE_SIZE = 1024
PLANE_EPS = 0.004
BOUND = 2.0
INV_EPS2 = 1.0 / (PLANE_EPS * PLANE_EPS)

NC, NS, L = 2, 16, 16
NW = NC * NS
N = 16384 * 256
PER_W = N // NW
CHUNK = 2048
N_CHUNKS = PER_W // CHUNK
NSTR = 4
SEG = CHUNK // NSTR


def _body(tmin_hbm, tmax_hbm, pos_hbm, w_hbm, out_hbm,
          pos_v0, pos_v1, pos_v2, pos_v3, w_v0, w_v1, w_v2, w_v3,
          zz_v0, zz_v1, idx_v0, idx_v1, gmin_v0, gmin_v1,
          gmax_v0, gmax_v1, out_v0, out_v1, isem, gsem, osem):
    pos_bufs = (pos_v0, pos_v1, pos_v2, pos_v3)
    w_bufs = (w_v0, w_v1, w_v2, w_v3)
    zz_bufs = (zz_v0, zz_v1)
    idx_bufs = (idx_v0, idx_v1)
    gmin_bufs = (gmin_v0, gmin_v1)
    gmax_bufs = (gmax_v0, gmax_v1)
    out_bufs = (out_v0, out_v1)
    wid = lax.axis_index("s") * NC + lax.axis_index("c")
    base_w = wid * PER_W
    iota = lax.iota(jnp.int32, L)

    def in_copies(c, q, sp):
        base = base_w + c * CHUNK
        return (pltpu.make_async_copy(
                    pos_hbm.at[pl.ds(base * 3, 3 * CHUNK)], pos_bufs[q],
                    isem.at[sp]),
                pltpu.make_async_copy(
                    w_hbm.at[pl.ds(base, CHUNK)], w_bufs[q], isem.at[sp]))

    def gather_copies(p):
        cps = []
        for s in range(NSTR):
            sl = pl.ds(s * SEG, SEG)
            cps.append(pltpu.make_async_copy(
                tmin_hbm.at[idx_bufs[p].at[sl]], gmin_bufs[p].at[sl],
                gsem.at[p]))
            cps.append(pltpu.make_async_copy(
                tmax_hbm.at[idx_bufs[p].at[sl]], gmax_bufs[p].at[sl],
                gsem.at[p]))
        return cps

    def out_copy(c, p):
        base = base_w + c * CHUNK
        return pltpu.make_async_copy(out_bufs[p],
                                     out_hbm.at[pl.ds(base, CHUNK)],
                                     osem.at[p])

    def pass_a(q, p):
        def body(i, carry):
            p3 = i * (3 * L) + iota * 3
            x = plsc.load_gather(pos_bufs[q], [p3])
            y = plsc.load_gather(pos_bufs[q], [p3 + 1])
            z = plsc.load_gather(pos_bufs[q], [p3 + 2])
            mag = jnp.maximum(jnp.maximum(jnp.abs(x), jnp.abs(y)), jnp.abs(z))
            r = 1.0 / jnp.maximum(mag, 1e-9)
            s = jnp.where(mag <= 1.0, 1.0, (2.0 - r) * r)
            cz = z * s
            fx = jnp.minimum((x * s + BOUND) * (PLANE_SIZE * 0.5 / BOUND),
                             PLANE_SIZE - 1.0)
            fy = jnp.minimum((y * s + BOUND) * (PLANE_SIZE * 0.5 / BOUND),
                             PLANE_SIZE - 1.0)
            xy = fx.astype(jnp.int32) * PLANE_SIZE + fy.astype(jnp.int32)
            zz_bufs[p][pl.ds(i * L, L)] = cz
            idx_bufs[p][pl.ds(i * L, L)] = xy
            return carry
        lax.fori_loop(0, CHUNK // L, body, None, unroll=2)

    def pass_b(qprev, pprev):
        def body(i, carry):
            zmin = gmin_bufs[pprev][pl.ds(i * L, L)]
            zmax = gmax_bufs[pprev][pl.ds(i * L, L)]
            zz = zz_bufs[pprev][pl.ds(i * L, L)]
            w = w_bufs[qprev][pl.ds(i * L, L)]
            wm = jnp.where(zz < zmin, 0.0, 1.0)
            wm = jnp.where(zz > zmax, 0.0, wm)
            d1 = zz - zmin
            d2 = zmax - zz
            q1 = d1 * d1 * INV_EPS2
            q2 = d2 * d2 * INV_EPS2
            i1 = (zz < zmin + PLANE_EPS) & (zz > zmin) & (zz < zmax - PLANE_EPS)
            i2 = (zz > zmax - PLANE_EPS) & (zz < zmax) & (zz > zmin + PLANE_EPS)
            wm = jnp.where(i1, q1, wm)
            wm = jnp.where(i2, q2, wm)
            out_bufs[pprev][pl.ds(i * L, L)] = w * wm
            return carry
        lax.fori_loop(0, CHUNK // L, body, None, unroll=2)

    def tail(cprev, qprev, pprev, wait_out):
        for a in gather_copies(pprev):
            a.wait()
        if wait_out is None:
            out_copy(cprev - 2, pprev).wait()
        else:
            pl.when(wait_out)(
                lambda: out_copy(jnp.maximum(cprev - 2, 0), pprev).wait())
        pass_b(qprev, pprev)
        out_copy(cprev, pprev).start()

    for a in in_copies(0, 0, 0):
        a.start()
    for a in in_copies(1, 1, 1):
        a.start()

    def quad(cc, carry):
        for q in range(4):
            c = 4 * cc + q
            p = q % 2
            for a in in_copies(c, q, p):
                a.wait()
            cnxt = jnp.minimum(c + 2, N_CHUNKS - 1)
            for a in in_copies(cnxt, (q + 2) % 4, p):
                a.start()
            pass_a(q, p)
            for a in gather_copies(p):
                a.start()
            if q == 0:
                pl.when(cc > 0)(
                    lambda: tail(jnp.maximum(c - 1, 0), 3, 1, None))
            elif q == 3:
                tail(c - 1, 2, 0, None)
            else:
                tail(c - 1, q - 1, (q - 1) % 2, cc > 0)
        return carry

    lax.fori_loop(0, N_CHUNKS // 4, quad, None)

    clast = N_CHUNKS - 1
    tail(clast, 3, 1, None)
    out_copy(clast - 1, 0).wait()
    out_copy(clast, 1).wait()
    for a in in_copies(N_CHUNKS - 1, 0, 0):
        a.wait()
    for a in in_copies(N_CHUNKS - 1, 1, 1):
        a.wait()


@jax.jit
def _occ_mask_sc(tmin, tmax, pos_flat, w_flat):
    mesh = plsc.VectorSubcoreMesh(core_axis_name="c", subcore_axis_name="s",
                                  num_cores=NC, num_subcores=NS)
    f = pl.kernel(
        _body,
        out_type=jax.ShapeDtypeStruct((N,), jnp.float32),
        mesh=mesh,
        compiler_params=pltpu.CompilerParams(needs_layout_passes=False),
        scratch_types=[
            pltpu.VMEM((3 * CHUNK,), jnp.float32),
            pltpu.VMEM((3 * CHUNK,), jnp.float32),
            pltpu.VMEM((3 * CHUNK,), jnp.float32),
            pltpu.VMEM((3 * CHUNK,), jnp.float32),
            pltpu.VMEM((CHUNK,), jnp.float32),
            pltpu.VMEM((CHUNK,), jnp.float32),
            pltpu.VMEM((CHUNK,), jnp.float32),
            pltpu.VMEM((CHUNK,), jnp.float32),
            pltpu.VMEM((CHUNK,), jnp.float32),
            pltpu.VMEM((CHUNK,), jnp.float32),
            pltpu.VMEM((CHUNK,), jnp.int32),
            pltpu.VMEM((CHUNK,), jnp.int32),
            pltpu.VMEM((CHUNK,), jnp.float32),
            pltpu.VMEM((CHUNK,), jnp.float32),
            pltpu.VMEM((CHUNK,), jnp.float32),
            pltpu.VMEM((CHUNK,), jnp.float32),
            pltpu.VMEM((CHUNK,), jnp.float32),
            pltpu.VMEM((CHUNK,), jnp.float32),
            pltpu.SemaphoreType.DMA((2,)),
            pltpu.SemaphoreType.DMA((2,)),
            pltpu.SemaphoreType.DMA((2,)),
        ],
    )
    return f(tmin, tmax, pos_flat, w_flat)


def kernel(positions, weights, occ_plane_min, occ_plane_max):
    prefix = positions.shape[:-1]
    out = _occ_mask_sc(occ_plane_min, occ_plane_max,
                       positions.reshape(-1), weights.reshape(-1))
    return out.reshape(prefix + (1,))
